# baseline (speedup 1.0000x reference)
import jax
import jax.numpy as jnp
import numpy as np
from jax import lax
from jax.experimental import pallas as pl
from jax.experimental.pallas import tpu as pltpu
from jax.experimental.pallas import tpu_sc as plsc

H = 8
Dh = 16
C = 128
N = 50000
N_P = 50048
E = 400000
PE = 8192

_NC = 2
_NS = 16

_EPAD = 409600
_ECH = _EPAD // _NS
_B1 = 256
_NB1 = _ECH // _B1
_JB1 = _B1 // 128
_B2 = 512
_NB2 = _ECH // _B2
_JB2 = _B2 // 128
_RPT = N_P // _NS
_ZROWS = 136

_f32 = jnp.float32
_i32 = jnp.int32


def _iota16():
    return lax.iota(_i32, 16)


def _edgeA_body(q_hbm, kv_hbm, s_hbm, d_hbm, msg_hbm, den_hbm,
                sbuf, dbuf, qidx, kvidx, q4, kv4, msg4, exbuf, zrow, accd):
    c = lax.axis_index("c")
    sid = lax.axis_index("s")

    zero16 = jnp.zeros((16,), _f32)

    def _ze(i, _):
        exbuf[i // 128, i % 128, pl.ds(0, 16)] = zero16
        return 0
    lax.fori_loop(0, _JB1 * 128, _ze, 0)

    def _zr(i, _):
        zrow[i, pl.ds(0, 16)] = zero16
        return 0
    lax.fori_loop(0, _ZROWS, _zr, 0)

    def _zacc(t, _):
        pltpu.sync_copy(zrow, accd.at[pl.ds(sid * _RPT + t * _ZROWS, _ZROWS)])
        return 0
    lax.fori_loop(0, _RPT // _ZROWS, _zacc, 0)
    plsc.subcore_barrier()

    def _batch(b, _):
        row0 = sid * (_ECH // 128) + b * _JB1
        pltpu.sync_copy(s_hbm.at[pl.ds(row0, _JB1)], sbuf)
        pltpu.sync_copy(d_hbm.at[pl.ds(row0, _JB1)], dbuf)

        for jj in range(_JB1):
            for g in range(8):
                sv = sbuf[jj, pl.ds(g * 16, 16)]
                dv = dbuf[jj, pl.ds(g * 16, 16)]
                kvidx[jj, pl.ds(g * 16, 16)] = sv * 2 + c
                qidx[jj, pl.ds(g * 16, 16)] = dv * 2 + c

        base_eg = sid * _ECH + b * _B1
        for j in range(_JB1):
            jv = jnp.full((16,), j, _i32)
            pltpu.sync_copy(q_hbm.at[qidx.at[j]], q4.at[j])
            pltpu.sync_copy(kv_hbm.at[kvidx.at[j]], kv4.at[j])

            def _grp(g, _):
                rr = g * 16 + _iota16()
                eg = base_eg + j * 128 + rr
                valid = eg < E
                for h in range(4):
                    pv = jnp.full((16,), (h // 2) * _JB1 + j, _i32)

                    prods = []
                    for j2 in range(16):
                        j2v = jnp.full((16,), j2, _i32)
                        qT = plsc.load_gather(q4, [jv, rr, j2v + 16 * h])
                        kT = plsc.load_gather(kv4, [jv, rr, j2v + 32 * h])
                        prods.append(qT * kT)
                    while len(prods) > 1:
                        prods = [prods[i] + prods[i + 1]
                                 for i in range(0, len(prods), 2)]
                    a = prods[0]

                    ex = jnp.where(valid, jnp.exp(jnp.minimum(a, 80.0)), 0.0)

                    coff = 32 * h + 16
                    moff = 16 * (h % 2)

                    for j2 in range(16):
                        j2v = jnp.full((16,), j2, _i32)
                        vT = plsc.load_gather(kv4, [jv, rr, j2v + coff])
                        plsc.store_scatter(msg4, [pv, rr, j2v + moff],
                                           vT * ex)
                    plsc.store_scatter(
                        exbuf, [jv, rr, jnp.full((16,), h, _i32)], ex)
                return 0
            lax.fori_loop(0, 8, _grp, 0)

            pltpu.sync_copy(exbuf.at[j], accd.at[dbuf.at[j]], add=True)
            erow0 = base_eg + j * 128
            for p in range(2):
                pltpu.sync_copy(msg4.at[p * _JB1 + j],
                                msg_hbm.at[c * 2 + p, pl.ds(erow0, 128)])
        return 0
    lax.fori_loop(0, _NB1, _batch, 0)

    plsc.subcore_barrier()
    pltpu.sync_copy(accd.at[pl.ds(sid * _RPT, _RPT)],
                    den_hbm.at[c, pl.ds(sid * _RPT, _RPT)])


def _edgeA(q_tab, kv_tab, s2d, d2d):
    mesh = plsc.VectorSubcoreMesh(core_axis_name="c", subcore_axis_name="s",
                                  num_cores=_NC, num_subcores=_NS)
    return pl.kernel(
        _edgeA_body,
        out_type=(jax.ShapeDtypeStruct((4, _EPAD, 32), _f32),
                  jax.ShapeDtypeStruct((_NC, N_P, 16), _f32)),
        mesh=mesh,
        compiler_params=pltpu.CompilerParams(
            needs_layout_passes=False, use_tc_tiling_on_sc=False),
        scratch_types=[
            pltpu.VMEM((_JB1, 128), _i32),
            pltpu.VMEM((_JB1, 128), _i32),
            pltpu.VMEM((_JB1, 128), _i32),
            pltpu.VMEM((_JB1, 128), _i32),
            pltpu.VMEM((_JB1, 128, 64), _f32),
            pltpu.VMEM((_JB1, 128, 128), _f32),
            pltpu.VMEM((2 * _JB1, 128, 32), _f32),
            pltpu.VMEM((_JB1, 128, 16), _f32),
            pltpu.VMEM((_ZROWS, 16), _f32),
            pltpu.VMEM_SHARED((N_P, 16), _f32),
        ],
    )(q_tab, kv_tab, s2d, d2d)


def _edgeB_body(msg_hbm, d_hbm, out_hbm, dbuf, mbuf, zrow, acc):
    c = lax.axis_index("c")
    sid = lax.axis_index("s")

    zero16 = jnp.zeros((16,), _f32)

    def _zr(i, _):
        zrow[i // 2, pl.ds((i % 2) * 16, 16)] = zero16
        return 0
    lax.fori_loop(0, 2 * _ZROWS, _zr, 0)

    for p in range(2):
        def _zacc(t, _):
            pltpu.sync_copy(zrow, acc.at[pl.ds(sid * _RPT + t * _ZROWS,
                                               _ZROWS)])
            return 0
        lax.fori_loop(0, _RPT // _ZROWS, _zacc, 0)
        plsc.subcore_barrier()

        def _batch(b, _):
            row0 = sid * (_ECH // 128) + b * _JB2
            pltpu.sync_copy(d_hbm.at[pl.ds(row0, _JB2)], dbuf)
            erow0 = sid * _ECH + b * _B2
            pltpu.sync_copy(msg_hbm.at[c * 2 + p, pl.ds(erow0, _B2)], mbuf)
            for j in range(_JB2):
                pltpu.sync_copy(mbuf.at[pl.ds(j * 128, 128)],
                                acc.at[dbuf.at[j]], add=True)
            return 0
        lax.fori_loop(0, _NB2, _batch, 0)

        plsc.subcore_barrier()
        pltpu.sync_copy(acc.at[pl.ds(sid * _RPT, _RPT)],
                        out_hbm.at[c * 2 + p, pl.ds(sid * _RPT, _RPT)])


def _edgeB(msg, d2d):
    mesh = plsc.VectorSubcoreMesh(core_axis_name="c", subcore_axis_name="s",
                                  num_cores=_NC, num_subcores=_NS)
    return pl.kernel(
        _edgeB_body,
        out_type=jax.ShapeDtypeStruct((4, N_P, 32), _f32),
        mesh=mesh,
        compiler_params=pltpu.CompilerParams(
            needs_layout_passes=False, use_tc_tiling_on_sc=False),
        scratch_types=[
            pltpu.VMEM((_JB2, 128), _i32),
            pltpu.VMEM((_B2, 32), _f32),
            pltpu.VMEM((_ZROWS, 32), _f32),
            pltpu.VMEM_SHARED((N_P, 32), _f32),
        ],
    )(msg, d2d)


def _lp_gather_body(embd_hbm, embr_hbm, idxd_hbm, idxr_hbm, zs_hbm, zd_hbm,
                    idxbuf, rowbuf):
    c = lax.axis_index("c")
    sid = lax.axis_index("s")
    wid = sid * _NC + c
    for t in range(4):
        ch = wid + t * 32
        pltpu.sync_copy(idxd_hbm.at[ch], idxbuf)
        pltpu.sync_copy(embd_hbm.at[idxbuf], rowbuf)
        pltpu.sync_copy(rowbuf, zs_hbm.at[pl.ds(ch * 128, 128)])
        pltpu.sync_copy(idxr_hbm.at[ch], idxbuf)
        pltpu.sync_copy(embr_hbm.at[idxbuf], rowbuf)
        pltpu.sync_copy(rowbuf, zd_hbm.at[pl.ds(ch * 128, 128)])


def _lp_gather(emb_dev, emb_repo, idx_dev2d, idx_repo2d):
    mesh = plsc.VectorSubcoreMesh(core_axis_name="c", subcore_axis_name="s",
                                  num_cores=_NC, num_subcores=_NS)
    return pl.kernel(
        _lp_gather_body,
        out_type=(jax.ShapeDtypeStruct((2 * PE, C), _f32),
                  jax.ShapeDtypeStruct((2 * PE, C), _f32)),
        mesh=mesh,
        compiler_params=pltpu.CompilerParams(
            needs_layout_passes=False, use_tc_tiling_on_sc=False),
        scratch_types=[
            pltpu.VMEM((128,), _i32),
            pltpu.VMEM((128, C), _f32),
        ],
    )(emb_dev, emb_repo, idx_dev2d, idx_repo2d)


_BN = 3128


def _proj_body(x_ref, wq_ref, bq_ref, wkv_ref, bkv_ref, q_ref, kv_ref):
    x = x_ref[...]
    q_ref[...] = jnp.dot(x, wq_ref[...], preferred_element_type=_f32) + bq_ref[...]
    kv_ref[...] = jnp.dot(x, wkv_ref[...], preferred_element_type=_f32) + bkv_ref[...]


def _proj(x, wq, bq, wkv, bkv):
    grid = (N_P // _BN,)
    return pl.pallas_call(
        _proj_body,
        grid=grid,
        in_specs=[
            pl.BlockSpec((_BN, C), lambda i: (i, 0)),
            pl.BlockSpec((C, C), lambda i: (0, 0)),
            pl.BlockSpec((1, C), lambda i: (0, 0)),
            pl.BlockSpec((C, 2 * C), lambda i: (0, 0)),
            pl.BlockSpec((1, 2 * C), lambda i: (0, 0)),
        ],
        out_specs=[
            pl.BlockSpec((_BN, C), lambda i: (i, 0)),
            pl.BlockSpec((_BN, 2 * C), lambda i: (i, 0)),
        ],
        out_shape=[
            jax.ShapeDtypeStruct((N_P, C), _f32),
            jax.ShapeDtypeStruct((N_P, 2 * C), _f32),
        ],
    )(x, wq, bq, wkv, bkv)


def _final_body(num_ref, den_ref, x_ref, wa_ref, ba_ref, beta_ref, o_ref):
    x = x_ref[...]
    tot = jnp.zeros((x.shape[0], C), _f32)
    for h in range(H):
        core, hp = h // 4, h % 4
        num = num_ref[core * 2 + hp // 2, :,
                      pl.ds((hp % 2) * 16, 16)]
        den = den_ref[core, :, pl.ds(hp, 1)]
        g = jax.nn.gelu(num / (den + 1e-30))
        tot = tot + jnp.dot(g, wa_ref[pl.ds(h * 16, 16), :],
                            preferred_element_type=_f32)
    beta = beta_ref[0, 0]
    y = beta * (tot + ba_ref[...]) + (1.0 - beta) * x
    o_ref[...] = jnp.where(y >= 0, y, 0.01 * y)


def _finalize(num, den, x_prev, wa, ba, beta):
    grid = (N_P // _BN,)
    return pl.pallas_call(
        _final_body,
        grid=grid,
        in_specs=[
            pl.BlockSpec((4, _BN, 32), lambda i: (0, i, 0)),
            pl.BlockSpec((2, _BN, 16), lambda i: (0, i, 0)),
            pl.BlockSpec((_BN, C), lambda i: (i, 0)),
            pl.BlockSpec((C, C), lambda i: (0, 0)),
            pl.BlockSpec((1, C), lambda i: (0, 0)),
            pl.BlockSpec((1, 1), lambda i: (0, 0), memory_space=pltpu.SMEM),
        ],
        out_specs=pl.BlockSpec((_BN, C), lambda i: (i, 0)),
        out_shape=jax.ShapeDtypeStruct((N_P, C), _f32),
    )(num, den, x_prev, wa, ba, beta)


def _fit_body(er_ref, el_ref, w2_ref, b2_ref, emb_ref, fit_ref):
    emb = er_ref[...] + el_ref[...]
    emb_ref[...] = emb
    fit_ref[...] = (jnp.sum(emb * w2_ref[...], axis=1, keepdims=True)
                    + b2_ref[0, 0])


def _fit(emb_res, emb_link, w2row, b2):
    grid = (N_P // _BN,)
    return pl.pallas_call(
        _fit_body,
        grid=grid,
        in_specs=[
            pl.BlockSpec((_BN, C), lambda i: (i, 0)),
            pl.BlockSpec((_BN, C), lambda i: (i, 0)),
            pl.BlockSpec((1, C), lambda i: (0, 0)),
            pl.BlockSpec((1, 1), lambda i: (0, 0), memory_space=pltpu.SMEM),
        ],
        out_specs=[
            pl.BlockSpec((_BN, C), lambda i: (i, 0)),
            pl.BlockSpec((_BN, 1), lambda i: (i, 0)),
        ],
        out_shape=[
            jax.ShapeDtypeStruct((N_P, C), _f32),
            jax.ShapeDtypeStruct((N_P, 1), _f32),
        ],
    )(emb_res, emb_link, w2row, b2)


def _score_body(zs_ref, zd_ref, ws_ref, bs_ref, wd_ref, bd_ref, bf_ref, o_ref):
    hs = jnp.dot(zs_ref[...], ws_ref[...], preferred_element_type=_f32) + bs_ref[...]
    hd = jnp.dot(zd_ref[...], wd_ref[...], preferred_element_type=_f32) + bd_ref[...]
    pre = jnp.sum(hs * hd, axis=1, keepdims=True) + bf_ref[0, 0]
    o_ref[...] = jax.nn.sigmoid(pre)


def _score(zs, zd, ws, bs, wd, bd, bf):
    m = 2 * PE
    bm = 2048
    grid = (m // bm,)
    return pl.pallas_call(
        _score_body,
        grid=grid,
        in_specs=[
            pl.BlockSpec((bm, C), lambda i: (i, 0)),
            pl.BlockSpec((bm, C), lambda i: (i, 0)),
            pl.BlockSpec((C, 2 * C), lambda i: (0, 0)),
            pl.BlockSpec((1, 2 * C), lambda i: (0, 0)),
            pl.BlockSpec((C, 2 * C), lambda i: (0, 0)),
            pl.BlockSpec((1, 2 * C), lambda i: (0, 0)),
            pl.BlockSpec((1, 1), lambda i: (0, 0), memory_space=pltpu.SMEM),
        ],
        out_specs=pl.BlockSpec((bm, 1), lambda i: (i, 0)),
        out_shape=jax.ShapeDtypeStruct((m, 1), _f32),
    )(zs, zd, ws, bs, wd, bd, bf)


def _fold_layer(p):
    out = {}
    for t, src_e, dst_e in (('dev', 'dev__repo', 'repo__dev'),
                            ('repo', 'repo__dev', 'dev__repo')):
        npar = p[t]
        scale = p[dst_e]['p'] / 4.0
        wq = (npar['Wq'].reshape(C, H, Dh) * scale[None, :, None]).reshape(C, C)
        bq = (npar['bq'].reshape(H, Dh) * scale[:, None]).reshape(1, C)
        att, msg = p[src_e]['att'], p[src_e]['msg']
        wk = jnp.einsum('chd,hde->che', npar['Wk'].reshape(C, H, Dh), att)
        bk = jnp.einsum('hd,hde->he', npar['bk'].reshape(H, Dh), att)
        wv = jnp.einsum('chd,hde->che', npar['Wv'].reshape(C, H, Dh), msg)
        bv = jnp.einsum('hd,hde->he', npar['bv'].reshape(H, Dh), msg)
        wkv = jnp.concatenate([wk, wv], axis=-1).reshape(C, 2 * C)
        bkv = jnp.concatenate([bk, bv], axis=-1).reshape(1, 2 * C)
        out[t] = dict(Wq=wq, bq=bq, Wkv=wkv, bkv=bkv, Wa=npar['Wa'],
                      ba=npar['ba'].reshape(1, C),
                      beta=jax.nn.sigmoid(npar['skip']).reshape(1, 1))
    return out


def kernel(x_dev, x_repo, edge_index_dev_repo, edge_index_repo_dev,
           pos_edge, neg_edge, params):
    pad = _EPAD - E
    s_dr = jnp.pad(edge_index_dev_repo[0], (0, pad)).reshape(-1, 128)
    d_dr = jnp.pad(edge_index_dev_repo[1], (0, pad)).reshape(-1, 128)
    s_rd = jnp.pad(edge_index_repo_dev[0], (0, pad)).reshape(-1, 128)
    d_rd = jnp.pad(edge_index_repo_dev[1], (0, pad)).reshape(-1, 128)

    xp_dev = jnp.pad(x_dev, ((0, N_P - N), (0, 0)))
    xp_repo = jnp.pad(x_repo, ((0, N_P - N), (0, 0)))

    def net(layers):
        xd = {'dev': xp_dev, 'repo': xp_repo}
        for p in layers:
            f = _fold_layer(p)
            q, kv = {}, {}
            for t in ('dev', 'repo'):
                qt, kvt = _proj(xd[t], f[t]['Wq'], f[t]['bq'],
                                f[t]['Wkv'], f[t]['bkv'])
                q[t] = qt.reshape(N_P * 2, 4 * Dh)
                kv[t] = kvt.reshape(N_P * 2, 8 * Dh)
            msg_r, den_repo = _edgeA(q['repo'], kv['dev'], s_dr, d_dr)
            num_repo = _edgeB(msg_r, d_dr)
            msg_d, den_dev = _edgeA(q['dev'], kv['repo'], s_rd, d_rd)
            num_dev = _edgeB(msg_d, d_rd)
            xd = {'dev': _finalize(num_dev, den_dev, xd['dev'],
                                   f['dev']['Wa'], f['dev']['ba'],
                                   f['dev']['beta']),
                  'repo': _finalize(num_repo, den_repo, xd['repo'],
                                    f['repo']['Wa'], f['repo']['ba'],
                                    f['repo']['beta'])}
        return xd

    er = net(params['res'])
    el = net(params['link'])

    reg = params['reg']
    w2 = (reg['lin']['W'] @ reg['regress']['W']).reshape(1, C)
    b2 = (reg['lin']['b'] @ reg['regress']['W']
          + reg['regress']['b']).reshape(1, 1)
    emb_dev, fit_dev = _fit(er['dev'], el['dev'], w2, b2)
    emb_repo, fit_repo = _fit(er['repo'], el['repo'], w2, b2)

    idx_dev = jnp.concatenate([pos_edge[0], neg_edge[0]]).reshape(128, 128)
    idx_repo = jnp.concatenate([pos_edge[1], neg_edge[1]]).reshape(128, 128)
    zs, zd = _lp_gather(emb_dev, emb_repo, idx_dev, idx_repo)

    lp = params['lp']
    wf = lp['final']['W'][:, 0]
    ws = lp['src']['W'] * wf[None, :]
    bs = (lp['src']['b'] * wf).reshape(1, 2 * C)
    wd = lp['dst']['W']
    bd = lp['dst']['b'].reshape(1, 2 * C)
    bf = lp['final']['b'].reshape(1, 1)
    score = _score(zs, zd, ws, bs, wd, bd, bf)

    return fit_dev[:N, 0], fit_repo[:N, 0], score[:, 0]


# speedup vs baseline: 1.0964x; 1.0964x over previous
import jax
import jax.numpy as jnp
import numpy as np
from jax import lax
from jax.experimental import pallas as pl
from jax.experimental.pallas import tpu as pltpu
from jax.experimental.pallas import tpu_sc as plsc

H = 8
Dh = 16
C = 128
N = 50000
N_P = 50048
E = 400000
PE = 8192

_NC = 2
_NS = 16

_EPAD = 409600
_ECH = _EPAD // _NS
_B1 = 128
_NB1 = _ECH // _B1
_B2 = 512
_NB2 = _ECH // _B2
_JB2 = _B2 // 128
_RPT = N_P // _NS
_ZROWS = 136

_f32 = jnp.float32
_i32 = jnp.int32


def _iota16():
    return lax.iota(_i32, 16)


def _edgeA_body(q_hbm, kv_hbm, s_hbm, d_hbm, msg_hbm, den_hbm,
                sbuf0, dbuf0, qidx0, kvidx0, dnode0, q40, kv40, msg40, ex0,
                sbuf1, dbuf1, qidx1, kvidx1, dnode1, q41, kv41, msg41, ex1,
                zrow, accd, si0, si1, sg0, sg1, so0, so1):
    c = lax.axis_index("c")
    sid = lax.axis_index("s")

    par = ((sbuf0, dbuf0, qidx0, kvidx0, dnode0, q40, kv40, msg40, ex0,
            si0, sg0, so0),
           (sbuf1, dbuf1, qidx1, kvidx1, dnode1, q41, kv41, msg41, ex1,
            si1, sg1, so1))

    zero16 = jnp.zeros((16,), _f32)

    def _ze(i, _):
        ex0[i, pl.ds(0, 16)] = zero16
        ex1[i, pl.ds(0, 16)] = zero16
        return 0
    lax.fori_loop(0, 128, _ze, 0)

    def _zr(i, _):
        zrow[i, pl.ds(0, 16)] = zero16
        return 0
    lax.fori_loop(0, _ZROWS, _zr, 0)

    def _zacc(t, _):
        pltpu.sync_copy(zrow, accd.at[pl.ds(sid * _RPT + t * _ZROWS, _ZROWS)])
        return 0
    lax.fori_loop(0, _RPT // _ZROWS, _zacc, 0)
    plsc.subcore_barrier()

    idrow0 = sid * _NB1

    def _issue_ids(b, p):
        sbuf, dbuf, si = par[p][0], par[p][1], par[p][9]
        rowc = idrow0 + jnp.minimum(b, _NB1 - 1)
        pltpu.async_copy(s_hbm.at[rowc], sbuf, si)
        pltpu.async_copy(d_hbm.at[rowc], dbuf, si)

    def _drain_ids(p):
        sbuf, dbuf, si = par[p][0], par[p][1], par[p][9]
        pltpu.make_async_copy(s_hbm.at[idrow0], sbuf, si).wait()
        pltpu.make_async_copy(d_hbm.at[idrow0], dbuf, si).wait()

    def _calc_idx(p):
        sbuf, dbuf, qidx, kvidx, dnode = par[p][:5]
        for g in range(8):
            sv = sbuf[pl.ds(g * 16, 16)]
            dv = dbuf[pl.ds(g * 16, 16)]
            kvidx[pl.ds(g * 16, 16)] = sv * 2 + c
            qidx[pl.ds(g * 16, 16)] = dv * 2 + c
            dnode[pl.ds(g * 16, 16)] = dv

    def _issue_gather(p):
        qidx, kvidx, q4, kv4, sg = (par[p][2], par[p][3], par[p][5],
                                    par[p][6], par[p][10])
        pltpu.async_copy(q_hbm.at[qidx], q4, sg)
        pltpu.async_copy(kv_hbm.at[kvidx], kv4, sg)

    def _drain_gather(p):
        qidx, kvidx, q4, kv4, sg = (par[p][2], par[p][3], par[p][5],
                                    par[p][6], par[p][10])
        pltpu.make_async_copy(q_hbm.at[qidx], q4, sg).wait()
        pltpu.make_async_copy(kv_hbm.at[kvidx], kv4, sg).wait()

    def _compute(b, p):
        dnode, q4, kv4, msg4, exbuf = (par[p][4], par[p][5], par[p][6],
                                       par[p][7], par[p][8])
        base_eg = sid * _ECH + b * _B1

        def _grp(g, _):
            rr = g * 16 + _iota16()
            eg = base_eg + rr
            valid = eg < E
            for h in range(4):
                pv = jnp.full((16,), h // 2, _i32)

                prods = []
                for j2 in range(16):
                    j2v = jnp.full((16,), j2, _i32)
                    qT = plsc.load_gather(q4, [rr, j2v + 16 * h])
                    kT = plsc.load_gather(kv4, [rr, j2v + 32 * h])
                    prods.append(qT * kT)
                while len(prods) > 1:
                    prods = [prods[i] + prods[i + 1]
                             for i in range(0, len(prods), 2)]

                ex = jnp.where(valid,
                               jnp.exp(jnp.minimum(prods[0], 80.0)), 0.0)

                coff = 32 * h + 16
                moff = 16 * (h % 2)
                for j2 in range(16):
                    j2v = jnp.full((16,), j2, _i32)
                    vT = plsc.load_gather(kv4, [rr, j2v + coff])
                    plsc.store_scatter(msg4, [pv, rr, j2v + moff], vT * ex)
                plsc.store_scatter(
                    exbuf, [rr, jnp.full((16,), h, _i32)], ex)
            return 0
        lax.fori_loop(0, 8, _grp, 0)

        pltpu.sync_copy(exbuf, accd.at[dnode], add=True)

    def _issue_out(b, p):
        msg4, so = par[p][7], par[p][11]
        erow0 = sid * _ECH + b * _B1
        for pp in range(2):
            pltpu.async_copy(msg4.at[pp],
                             msg_hbm.at[c * 2 + pp, pl.ds(erow0, 128)], so)

    def _drain_out(p):
        msg4, so = par[p][7], par[p][11]
        for pp in range(2):
            pltpu.make_async_copy(
                msg4.at[pp],
                msg_hbm.at[c * 2 + pp, pl.ds(sid * _ECH, 128)], so).wait()

    def _step(b, p, first):
        if not first:
            _drain_out(p)
        _drain_gather(p)
        _compute(b, p)
        _issue_out(b, p)
        _drain_ids(1 - p)
        _calc_idx(1 - p)
        _issue_gather(1 - p)
        _issue_ids(b + 2, p)

    _issue_ids(0, 0)
    _issue_ids(1, 1)
    _drain_ids(0)
    _calc_idx(0)
    _issue_gather(0)
    _step(0, 0, True)
    _step(1, 1, True)

    def _loop(bb, _):
        b = 2 * bb
        _step(b, 0, False)
        _step(b + 1, 1, False)
        return 0
    lax.fori_loop(1, _NB1 // 2, _loop, 0)

    _drain_out(0)
    _drain_out(1)
    _drain_gather(0)
    _drain_ids(1)

    plsc.subcore_barrier()
    pltpu.sync_copy(accd.at[pl.ds(sid * _RPT, _RPT)],
                    den_hbm.at[c, pl.ds(sid * _RPT, _RPT)])


def _edgeA(q_tab, kv_tab, s2d, d2d):
    mesh = plsc.VectorSubcoreMesh(core_axis_name="c", subcore_axis_name="s",
                                  num_cores=_NC, num_subcores=_NS)
    return pl.kernel(
        _edgeA_body,
        out_type=(jax.ShapeDtypeStruct((4, _EPAD, 32), _f32),
                  jax.ShapeDtypeStruct((_NC, N_P, 16), _f32)),
        mesh=mesh,
        compiler_params=pltpu.CompilerParams(
            needs_layout_passes=False, use_tc_tiling_on_sc=False),
        scratch_types=[
            pltpu.VMEM((128,), _i32),
            pltpu.VMEM((128,), _i32),
            pltpu.VMEM((128,), _i32),
            pltpu.VMEM((128,), _i32),
            pltpu.VMEM((128,), _i32),
            pltpu.VMEM((128, 64), _f32),
            pltpu.VMEM((128, 128), _f32),
            pltpu.VMEM((2, 128, 32), _f32),
            pltpu.VMEM((128, 16), _f32),
            pltpu.VMEM((128,), _i32),
            pltpu.VMEM((128,), _i32),
            pltpu.VMEM((128,), _i32),
            pltpu.VMEM((128,), _i32),
            pltpu.VMEM((128,), _i32),
            pltpu.VMEM((128, 64), _f32),
            pltpu.VMEM((128, 128), _f32),
            pltpu.VMEM((2, 128, 32), _f32),
            pltpu.VMEM((128, 16), _f32),
            pltpu.VMEM((_ZROWS, 16), _f32),
            pltpu.VMEM_SHARED((N_P, 16), _f32),
            pltpu.SemaphoreType.DMA,
            pltpu.SemaphoreType.DMA,
            pltpu.SemaphoreType.DMA,
            pltpu.SemaphoreType.DMA,
            pltpu.SemaphoreType.DMA,
            pltpu.SemaphoreType.DMA,
        ],
    )(q_tab, kv_tab, s2d, d2d)


def _edgeB_body(msg_hbm, d_hbm, out_hbm, dbuf, mbuf, zrow, acc):
    c = lax.axis_index("c")
    sid = lax.axis_index("s")

    zero16 = jnp.zeros((16,), _f32)

    def _zr(i, _):
        zrow[i // 2, pl.ds((i % 2) * 16, 16)] = zero16
        return 0
    lax.fori_loop(0, 2 * _ZROWS, _zr, 0)

    for p in range(2):
        def _zacc(t, _):
            pltpu.sync_copy(zrow, acc.at[pl.ds(sid * _RPT + t * _ZROWS,
                                               _ZROWS)])
            return 0
        lax.fori_loop(0, _RPT // _ZROWS, _zacc, 0)
        plsc.subcore_barrier()

        def _batch(b, _):
            row0 = sid * (_ECH // 128) + b * _JB2
            pltpu.sync_copy(d_hbm.at[pl.ds(row0, _JB2)], dbuf)
            erow0 = sid * _ECH + b * _B2
            pltpu.sync_copy(msg_hbm.at[c * 2 + p, pl.ds(erow0, _B2)], mbuf)
            for j in range(_JB2):
                pltpu.sync_copy(mbuf.at[pl.ds(j * 128, 128)],
                                acc.at[dbuf.at[j]], add=True)
            return 0
        lax.fori_loop(0, _NB2, _batch, 0)

        plsc.subcore_barrier()
        pltpu.sync_copy(acc.at[pl.ds(sid * _RPT, _RPT)],
                        out_hbm.at[c * 2 + p, pl.ds(sid * _RPT, _RPT)])


def _edgeB(msg, d2d):
    mesh = plsc.VectorSubcoreMesh(core_axis_name="c", subcore_axis_name="s",
                                  num_cores=_NC, num_subcores=_NS)
    return pl.kernel(
        _edgeB_body,
        out_type=jax.ShapeDtypeStruct((4, N_P, 32), _f32),
        mesh=mesh,
        compiler_params=pltpu.CompilerParams(
            needs_layout_passes=False, use_tc_tiling_on_sc=False),
        scratch_types=[
            pltpu.VMEM((_JB2, 128), _i32),
            pltpu.VMEM((_B2, 32), _f32),
            pltpu.VMEM((_ZROWS, 32), _f32),
            pltpu.VMEM_SHARED((N_P, 32), _f32),
        ],
    )(msg, d2d)


def _lp_gather_body(embd_hbm, embr_hbm, idxd_hbm, idxr_hbm, zs_hbm, zd_hbm,
                    idxbuf, rowbuf):
    c = lax.axis_index("c")
    sid = lax.axis_index("s")
    wid = sid * _NC + c
    for t in range(4):
        ch = wid + t * 32
        pltpu.sync_copy(idxd_hbm.at[ch], idxbuf)
        pltpu.sync_copy(embd_hbm.at[idxbuf], rowbuf)
        pltpu.sync_copy(rowbuf, zs_hbm.at[pl.ds(ch * 128, 128)])
        pltpu.sync_copy(idxr_hbm.at[ch], idxbuf)
        pltpu.sync_copy(embr_hbm.at[idxbuf], rowbuf)
        pltpu.sync_copy(rowbuf, zd_hbm.at[pl.ds(ch * 128, 128)])


def _lp_gather(emb_dev, emb_repo, idx_dev2d, idx_repo2d):
    mesh = plsc.VectorSubcoreMesh(core_axis_name="c", subcore_axis_name="s",
                                  num_cores=_NC, num_subcores=_NS)
    return pl.kernel(
        _lp_gather_body,
        out_type=(jax.ShapeDtypeStruct((2 * PE, C), _f32),
                  jax.ShapeDtypeStruct((2 * PE, C), _f32)),
        mesh=mesh,
        compiler_params=pltpu.CompilerParams(
            needs_layout_passes=False, use_tc_tiling_on_sc=False),
        scratch_types=[
            pltpu.VMEM((128,), _i32),
            pltpu.VMEM((128, C), _f32),
        ],
    )(emb_dev, emb_repo, idx_dev2d, idx_repo2d)


_BN = 3128


def _proj_body(x_ref, wq_ref, bq_ref, wkv_ref, bkv_ref, q_ref, kv_ref):
    x = x_ref[...]
    q_ref[...] = jnp.dot(x, wq_ref[...], preferred_element_type=_f32) + bq_ref[...]
    kv_ref[...] = jnp.dot(x, wkv_ref[...], preferred_element_type=_f32) + bkv_ref[...]


def _proj(x, wq, bq, wkv, bkv):
    grid = (N_P // _BN,)
    return pl.pallas_call(
        _proj_body,
        grid=grid,
        in_specs=[
            pl.BlockSpec((_BN, C), lambda i: (i, 0)),
            pl.BlockSpec((C, C), lambda i: (0, 0)),
            pl.BlockSpec((1, C), lambda i: (0, 0)),
            pl.BlockSpec((C, 2 * C), lambda i: (0, 0)),
            pl.BlockSpec((1, 2 * C), lambda i: (0, 0)),
        ],
        out_specs=[
            pl.BlockSpec((_BN, C), lambda i: (i, 0)),
            pl.BlockSpec((_BN, 2 * C), lambda i: (i, 0)),
        ],
        out_shape=[
            jax.ShapeDtypeStruct((N_P, C), _f32),
            jax.ShapeDtypeStruct((N_P, 2 * C), _f32),
        ],
    )(x, wq, bq, wkv, bkv)


def _final_body(num_ref, den_ref, x_ref, wa_ref, ba_ref, beta_ref, o_ref):
    x = x_ref[...]
    tot = jnp.zeros((x.shape[0], C), _f32)
    for h in range(H):
        core, hp = h // 4, h % 4
        num = num_ref[core * 2 + hp // 2, :,
                      pl.ds((hp % 2) * 16, 16)]
        den = den_ref[core, :, pl.ds(hp, 1)]
        g = jax.nn.gelu(num / (den + 1e-30))
        tot = tot + jnp.dot(g, wa_ref[pl.ds(h * 16, 16), :],
                            preferred_element_type=_f32)
    beta = beta_ref[0, 0]
    y = beta * (tot + ba_ref[...]) + (1.0 - beta) * x
    o_ref[...] = jnp.where(y >= 0, y, 0.01 * y)


def _finalize(num, den, x_prev, wa, ba, beta):
    grid = (N_P // _BN,)
    return pl.pallas_call(
        _final_body,
        grid=grid,
        in_specs=[
            pl.BlockSpec((4, _BN, 32), lambda i: (0, i, 0)),
            pl.BlockSpec((2, _BN, 16), lambda i: (0, i, 0)),
            pl.BlockSpec((_BN, C), lambda i: (i, 0)),
            pl.BlockSpec((C, C), lambda i: (0, 0)),
            pl.BlockSpec((1, C), lambda i: (0, 0)),
            pl.BlockSpec((1, 1), lambda i: (0, 0), memory_space=pltpu.SMEM),
        ],
        out_specs=pl.BlockSpec((_BN, C), lambda i: (i, 0)),
        out_shape=jax.ShapeDtypeStruct((N_P, C), _f32),
    )(num, den, x_prev, wa, ba, beta)


def _fit_body(er_ref, el_ref, w2_ref, b2_ref, emb_ref, fit_ref):
    emb = er_ref[...] + el_ref[...]
    emb_ref[...] = emb
    fit_ref[...] = (jnp.sum(emb * w2_ref[...], axis=1, keepdims=True)
                    + b2_ref[0, 0])


def _fit(emb_res, emb_link, w2row, b2):
    grid = (N_P // _BN,)
    return pl.pallas_call(
        _fit_body,
        grid=grid,
        in_specs=[
            pl.BlockSpec((_BN, C), lambda i: (i, 0)),
            pl.BlockSpec((_BN, C), lambda i: (i, 0)),
            pl.BlockSpec((1, C), lambda i: (0, 0)),
            pl.BlockSpec((1, 1), lambda i: (0, 0), memory_space=pltpu.SMEM),
        ],
        out_specs=[
            pl.BlockSpec((_BN, C), lambda i: (i, 0)),
            pl.BlockSpec((_BN, 1), lambda i: (i, 0)),
        ],
        out_shape=[
            jax.ShapeDtypeStruct((N_P, C), _f32),
            jax.ShapeDtypeStruct((N_P, 1), _f32),
        ],
    )(emb_res, emb_link, w2row, b2)


def _score_body(zs_ref, zd_ref, ws_ref, bs_ref, wd_ref, bd_ref, bf_ref, o_ref):
    hs = jnp.dot(zs_ref[...], ws_ref[...], preferred_element_type=_f32) + bs_ref[...]
    hd = jnp.dot(zd_ref[...], wd_ref[...], preferred_element_type=_f32) + bd_ref[...]
    pre = jnp.sum(hs * hd, axis=1, keepdims=True) + bf_ref[0, 0]
    o_ref[...] = jax.nn.sigmoid(pre)


def _score(zs, zd, ws, bs, wd, bd, bf):
    m = 2 * PE
    bm = 2048
    grid = (m // bm,)
    return pl.pallas_call(
        _score_body,
        grid=grid,
        in_specs=[
            pl.BlockSpec((bm, C), lambda i: (i, 0)),
            pl.BlockSpec((bm, C), lambda i: (i, 0)),
            pl.BlockSpec((C, 2 * C), lambda i: (0, 0)),
            pl.BlockSpec((1, 2 * C), lambda i: (0, 0)),
            pl.BlockSpec((C, 2 * C), lambda i: (0, 0)),
            pl.BlockSpec((1, 2 * C), lambda i: (0, 0)),
            pl.BlockSpec((1, 1), lambda i: (0, 0), memory_space=pltpu.SMEM),
        ],
        out_specs=pl.BlockSpec((bm, 1), lambda i: (i, 0)),
        out_shape=jax.ShapeDtypeStruct((m, 1), _f32),
    )(zs, zd, ws, bs, wd, bd, bf)


def _fold_layer(p):
    out = {}
    for t, src_e, dst_e in (('dev', 'dev__repo', 'repo__dev'),
                            ('repo', 'repo__dev', 'dev__repo')):
        npar = p[t]
        scale = p[dst_e]['p'] / 4.0
        wq = (npar['Wq'].reshape(C, H, Dh) * scale[None, :, None]).reshape(C, C)
        bq = (npar['bq'].reshape(H, Dh) * scale[:, None]).reshape(1, C)
        att, msg = p[src_e]['att'], p[src_e]['msg']
        wk = jnp.einsum('chd,hde->che', npar['Wk'].reshape(C, H, Dh), att)
        bk = jnp.einsum('hd,hde->he', npar['bk'].reshape(H, Dh), att)
        wv = jnp.einsum('chd,hde->che', npar['Wv'].reshape(C, H, Dh), msg)
        bv = jnp.einsum('hd,hde->he', npar['bv'].reshape(H, Dh), msg)
        wkv = jnp.concatenate([wk, wv], axis=-1).reshape(C, 2 * C)
        bkv = jnp.concatenate([bk, bv], axis=-1).reshape(1, 2 * C)
        out[t] = dict(Wq=wq, bq=bq, Wkv=wkv, bkv=bkv, Wa=npar['Wa'],
                      ba=npar['ba'].reshape(1, C),
                      beta=jax.nn.sigmoid(npar['skip']).reshape(1, 1))
    return out


def kernel(x_dev, x_repo, edge_index_dev_repo, edge_index_repo_dev,
           pos_edge, neg_edge, params):
    pad = _EPAD - E
    s_dr = jnp.pad(edge_index_dev_repo[0], (0, pad)).reshape(-1, 128)
    d_dr = jnp.pad(edge_index_dev_repo[1], (0, pad)).reshape(-1, 128)
    s_rd = jnp.pad(edge_index_repo_dev[0], (0, pad)).reshape(-1, 128)
    d_rd = jnp.pad(edge_index_repo_dev[1], (0, pad)).reshape(-1, 128)

    xp_dev = jnp.pad(x_dev, ((0, N_P - N), (0, 0)))
    xp_repo = jnp.pad(x_repo, ((0, N_P - N), (0, 0)))

    def net(layers):
        xd = {'dev': xp_dev, 'repo': xp_repo}
        for p in layers:
            f = _fold_layer(p)
            q, kv = {}, {}
            for t in ('dev', 'repo'):
                qt, kvt = _proj(xd[t], f[t]['Wq'], f[t]['bq'],
                                f[t]['Wkv'], f[t]['bkv'])
                q[t] = qt.reshape(N_P * 2, 4 * Dh)
                kv[t] = kvt.reshape(N_P * 2, 8 * Dh)
            msg_r, den_repo = _edgeA(q['repo'], kv['dev'], s_dr, d_dr)
            num_repo = _edgeB(msg_r, d_dr)
            msg_d, den_dev = _edgeA(q['dev'], kv['repo'], s_rd, d_rd)
            num_dev = _edgeB(msg_d, d_rd)
            xd = {'dev': _finalize(num_dev, den_dev, xd['dev'],
                                   f['dev']['Wa'], f['dev']['ba'],
                                   f['dev']['beta']),
                  'repo': _finalize(num_repo, den_repo, xd['repo'],
                                    f['repo']['Wa'], f['repo']['ba'],
                                    f['repo']['beta'])}
        return xd

    er = net(params['res'])
    el = net(params['link'])

    reg = params['reg']
    w2 = (reg['lin']['W'] @ reg['regress']['W']).reshape(1, C)
    b2 = (reg['lin']['b'] @ reg['regress']['W']
          + reg['regress']['b']).reshape(1, 1)
    emb_dev, fit_dev = _fit(er['dev'], el['dev'], w2, b2)
    emb_repo, fit_repo = _fit(er['repo'], el['repo'], w2, b2)

    idx_dev = jnp.concatenate([pos_edge[0], neg_edge[0]]).reshape(128, 128)
    idx_repo = jnp.concatenate([pos_edge[1], neg_edge[1]]).reshape(128, 128)
    zs, zd = _lp_gather(emb_dev, emb_repo, idx_dev, idx_repo)

    lp = params['lp']
    wf = lp['final']['W'][:, 0]
    ws = lp['src']['W'] * wf[None, :]
    bs = (lp['src']['b'] * wf).reshape(1, 2 * C)
    wd = lp['dst']['W']
    bd = lp['dst']['b'].reshape(1, 2 * C)
    bf = lp['final']['b'].reshape(1, 1)
    score = _score(zs, zd, ws, bs, wd, bd, bf)

    return fit_dev[:N, 0], fit_repo[:N, 0], score[:, 0]


# speedup vs baseline: 1.3107x; 1.1954x over previous
import jax
import jax.numpy as jnp
import numpy as np
from jax import lax
from jax.experimental import pallas as pl
from jax.experimental.pallas import tpu as pltpu
from jax.experimental.pallas import tpu_sc as plsc

H = 8
Dh = 16
C = 128
N = 50000
N_P = 50048
E = 400000
PE = 8192

_NC = 2
_NS = 16

_EPAD = 409600
_ECH = _EPAD // _NS
_B1 = 128
_NB1 = _ECH // _B1
_B2 = 512
_NB2 = _ECH // _B2
_JB2 = _B2 // 128
_RPT = N_P // _NS
_ZROWS = 136

_f32 = jnp.float32
_i32 = jnp.int32


def _iota16():
    return lax.iota(_i32, 16)


def _edgeA_body(q_hbm, kv_hbm, s_hbm, d_hbm, msg_hbm, den_hbm,
                sbuf0, dbuf0, qidx0, kvidx0, dnode0, q40, kv40, msg40, ex0,
                sbuf1, dbuf1, qidx1, kvidx1, dnode1, q41, kv41, msg41, ex1,
                zrow, accd, si0, si1, sg0, sg1, so0, so1):
    c = lax.axis_index("c")
    sid = lax.axis_index("s")

    par = ((sbuf0, dbuf0, qidx0, kvidx0, dnode0, q40, kv40, msg40, ex0,
            si0, sg0, so0),
           (sbuf1, dbuf1, qidx1, kvidx1, dnode1, q41, kv41, msg41, ex1,
            si1, sg1, so1))

    zero16 = jnp.zeros((16,), _f32)

    def _ze(i, _):
        ex0[i, pl.ds(0, 16)] = zero16
        ex1[i, pl.ds(0, 16)] = zero16
        return 0
    lax.fori_loop(0, 128, _ze, 0)

    def _zr(i, _):
        zrow[i, pl.ds(0, 16)] = zero16
        return 0
    lax.fori_loop(0, _ZROWS, _zr, 0)

    def _zacc(t, _):
        pltpu.sync_copy(zrow, accd.at[pl.ds(sid * _RPT + t * _ZROWS, _ZROWS)])
        return 0
    lax.fori_loop(0, _RPT // _ZROWS, _zacc, 0)
    plsc.subcore_barrier()

    idrow0 = sid * _NB1

    def _issue_ids(b, p):
        sbuf, dbuf, si = par[p][0], par[p][1], par[p][9]
        rowc = idrow0 + jnp.minimum(b, _NB1 - 1)
        pltpu.async_copy(s_hbm.at[rowc], sbuf, si)
        pltpu.async_copy(d_hbm.at[rowc], dbuf, si)

    def _drain_ids(p):
        sbuf, dbuf, si = par[p][0], par[p][1], par[p][9]
        pltpu.make_async_copy(s_hbm.at[idrow0], sbuf, si).wait()
        pltpu.make_async_copy(d_hbm.at[idrow0], dbuf, si).wait()

    def _calc_idx(p):
        sbuf, dbuf, qidx, kvidx, dnode = par[p][:5]
        for g in range(8):
            sv = sbuf[pl.ds(g * 16, 16)]
            dv = dbuf[pl.ds(g * 16, 16)]
            kvidx[pl.ds(g * 16, 16)] = sv * 2 + c
            qidx[pl.ds(g * 16, 16)] = dv * 2 + c
            dnode[pl.ds(g * 16, 16)] = dv

    def _issue_gather(p):
        qidx, kvidx, q4, kv4, sg = (par[p][2], par[p][3], par[p][5],
                                    par[p][6], par[p][10])
        pltpu.async_copy(q_hbm.at[qidx], q4, sg)
        pltpu.async_copy(kv_hbm.at[kvidx], kv4, sg)

    def _drain_gather(p):
        qidx, kvidx, q4, kv4, sg = (par[p][2], par[p][3], par[p][5],
                                    par[p][6], par[p][10])
        pltpu.make_async_copy(q_hbm.at[qidx], q4, sg).wait()
        pltpu.make_async_copy(kv_hbm.at[kvidx], kv4, sg).wait()

    def _compute(b, p):
        dnode, q4, kv4, msg4, exbuf = (par[p][4], par[p][5], par[p][6],
                                       par[p][7], par[p][8])
        base_eg = sid * _ECH + b * _B1

        def _grp(g, _):
            rr = g * 16 + _iota16()
            eg = base_eg + rr
            valid = eg < E
            for h in range(4):
                pv = jnp.full((16,), h // 2, _i32)

                prods = []
                for j2 in range(16):
                    j2v = jnp.full((16,), j2, _i32)
                    qT = plsc.load_gather(q4, [rr, j2v + 16 * h])
                    kT = plsc.load_gather(kv4, [rr, j2v + 32 * h])
                    prods.append(qT * kT)
                while len(prods) > 1:
                    prods = [prods[i] + prods[i + 1]
                             for i in range(0, len(prods), 2)]

                ex = jnp.where(valid,
                               jnp.exp(jnp.minimum(prods[0], 80.0)), 0.0)

                coff = 32 * h + 16
                moff = 16 * (h % 2)
                for j2 in range(16):
                    j2v = jnp.full((16,), j2, _i32)
                    vT = plsc.load_gather(kv4, [rr, j2v + coff])
                    plsc.store_scatter(msg4, [pv, rr, j2v + moff], vT * ex)
                plsc.store_scatter(
                    exbuf, [rr, jnp.full((16,), h, _i32)], ex)
            return 0
        lax.fori_loop(0, 8, _grp, 0)

        pltpu.sync_copy(exbuf, accd.at[dnode], add=True)

    def _issue_out(b, p):
        msg4, so = par[p][7], par[p][11]
        erow0 = sid * _ECH + b * _B1
        for pp in range(2):
            pltpu.async_copy(msg4.at[pp],
                             msg_hbm.at[c * 2 + pp, pl.ds(erow0, 128)], so)

    def _drain_out(p):
        msg4, so = par[p][7], par[p][11]
        for pp in range(2):
            pltpu.make_async_copy(
                msg4.at[pp],
                msg_hbm.at[c * 2 + pp, pl.ds(sid * _ECH, 128)], so).wait()

    def _step(b, p, first):
        if not first:
            _drain_out(p)
        _drain_ids(1 - p)
        _calc_idx(1 - p)
        _issue_gather(1 - p)
        _issue_ids(b + 2, p)
        _drain_gather(p)
        _compute(b, p)
        _issue_out(b, p)

    _issue_ids(0, 0)
    _issue_ids(1, 1)
    _drain_ids(0)
    _calc_idx(0)
    _issue_gather(0)
    _step(0, 0, True)
    _step(1, 1, True)

    def _loop(bb, _):
        b = 2 * bb
        _step(b, 0, False)
        _step(b + 1, 1, False)
        return 0
    lax.fori_loop(1, _NB1 // 2, _loop, 0)

    _drain_out(0)
    _drain_out(1)
    _drain_gather(0)
    _drain_ids(1)

    plsc.subcore_barrier()
    pltpu.sync_copy(accd.at[pl.ds(sid * _RPT, _RPT)],
                    den_hbm.at[c, pl.ds(sid * _RPT, _RPT)])


def _edgeA(q_tab, kv_tab, s2d, d2d):
    mesh = plsc.VectorSubcoreMesh(core_axis_name="c", subcore_axis_name="s",
                                  num_cores=_NC, num_subcores=_NS)
    return pl.kernel(
        _edgeA_body,
        out_type=(jax.ShapeDtypeStruct((4, _EPAD, 32), _f32),
                  jax.ShapeDtypeStruct((_NC, N_P, 16), _f32)),
        mesh=mesh,
        compiler_params=pltpu.CompilerParams(
            needs_layout_passes=False, use_tc_tiling_on_sc=False),
        scratch_types=[
            pltpu.VMEM((128,), _i32),
            pltpu.VMEM((128,), _i32),
            pltpu.VMEM((128,), _i32),
            pltpu.VMEM((128,), _i32),
            pltpu.VMEM((128,), _i32),
            pltpu.VMEM((128, 64), _f32),
            pltpu.VMEM((128, 128), _f32),
            pltpu.VMEM((2, 128, 32), _f32),
            pltpu.VMEM((128, 16), _f32),
            pltpu.VMEM((128,), _i32),
            pltpu.VMEM((128,), _i32),
            pltpu.VMEM((128,), _i32),
            pltpu.VMEM((128,), _i32),
            pltpu.VMEM((128,), _i32),
            pltpu.VMEM((128, 64), _f32),
            pltpu.VMEM((128, 128), _f32),
            pltpu.VMEM((2, 128, 32), _f32),
            pltpu.VMEM((128, 16), _f32),
            pltpu.VMEM((_ZROWS, 16), _f32),
            pltpu.VMEM_SHARED((N_P, 16), _f32),
            pltpu.SemaphoreType.DMA,
            pltpu.SemaphoreType.DMA,
            pltpu.SemaphoreType.DMA,
            pltpu.SemaphoreType.DMA,
            pltpu.SemaphoreType.DMA,
            pltpu.SemaphoreType.DMA,
        ],
    )(q_tab, kv_tab, s2d, d2d)


def _edgeB_body(msg_hbm, d_hbm, out_hbm, dbuf, mbuf, zrow, acc):
    c = lax.axis_index("c")
    sid = lax.axis_index("s")

    zero16 = jnp.zeros((16,), _f32)

    def _zr(i, _):
        zrow[i // 2, pl.ds((i % 2) * 16, 16)] = zero16
        return 0
    lax.fori_loop(0, 2 * _ZROWS, _zr, 0)

    for p in range(2):
        def _zacc(t, _):
            pltpu.sync_copy(zrow, acc.at[pl.ds(sid * _RPT + t * _ZROWS,
                                               _ZROWS)])
            return 0
        lax.fori_loop(0, _RPT // _ZROWS, _zacc, 0)
        plsc.subcore_barrier()

        def _batch(b, _):
            row0 = sid * (_ECH // 128) + b * _JB2
            pltpu.sync_copy(d_hbm.at[pl.ds(row0, _JB2)], dbuf)
            erow0 = sid * _ECH + b * _B2
            pltpu.sync_copy(msg_hbm.at[c * 2 + p, pl.ds(erow0, _B2)], mbuf)
            for j in range(_JB2):
                pltpu.sync_copy(mbuf.at[pl.ds(j * 128, 128)],
                                acc.at[dbuf.at[j]], add=True)
            return 0
        lax.fori_loop(0, _NB2, _batch, 0)

        plsc.subcore_barrier()
        pltpu.sync_copy(acc.at[pl.ds(sid * _RPT, _RPT)],
                        out_hbm.at[c * 2 + p, pl.ds(sid * _RPT, _RPT)])


def _edgeB(msg, d2d):
    mesh = plsc.VectorSubcoreMesh(core_axis_name="c", subcore_axis_name="s",
                                  num_cores=_NC, num_subcores=_NS)
    return pl.kernel(
        _edgeB_body,
        out_type=jax.ShapeDtypeStruct((4, N_P, 32), _f32),
        mesh=mesh,
        compiler_params=pltpu.CompilerParams(
            needs_layout_passes=False, use_tc_tiling_on_sc=False),
        scratch_types=[
            pltpu.VMEM((_JB2, 128), _i32),
            pltpu.VMEM((_B2, 32), _f32),
            pltpu.VMEM((_ZROWS, 32), _f32),
            pltpu.VMEM_SHARED((N_P, 32), _f32),
        ],
    )(msg, d2d)


def _lp_gather_body(embd_hbm, embr_hbm, idxd_hbm, idxr_hbm, zs_hbm, zd_hbm,
                    idxbuf, rowbuf):
    c = lax.axis_index("c")
    sid = lax.axis_index("s")
    wid = sid * _NC + c
    for t in range(4):
        ch = wid + t * 32
        pltpu.sync_copy(idxd_hbm.at[ch], idxbuf)
        pltpu.sync_copy(embd_hbm.at[idxbuf], rowbuf)
        pltpu.sync_copy(rowbuf, zs_hbm.at[pl.ds(ch * 128, 128)])
        pltpu.sync_copy(idxr_hbm.at[ch], idxbuf)
        pltpu.sync_copy(embr_hbm.at[idxbuf], rowbuf)
        pltpu.sync_copy(rowbuf, zd_hbm.at[pl.ds(ch * 128, 128)])


def _lp_gather(emb_dev, emb_repo, idx_dev2d, idx_repo2d):
    mesh = plsc.VectorSubcoreMesh(core_axis_name="c", subcore_axis_name="s",
                                  num_cores=_NC, num_subcores=_NS)
    return pl.kernel(
        _lp_gather_body,
        out_type=(jax.ShapeDtypeStruct((2 * PE, C), _f32),
                  jax.ShapeDtypeStruct((2 * PE, C), _f32)),
        mesh=mesh,
        compiler_params=pltpu.CompilerParams(
            needs_layout_passes=False, use_tc_tiling_on_sc=False),
        scratch_types=[
            pltpu.VMEM((128,), _i32),
            pltpu.VMEM((128, C), _f32),
        ],
    )(emb_dev, emb_repo, idx_dev2d, idx_repo2d)


_BN = 3128


def _proj_body(x_ref, wq_ref, bq_ref, wkv_ref, bkv_ref, q_ref, kv_ref):
    x = x_ref[...]
    q_ref[...] = jnp.dot(x, wq_ref[...], preferred_element_type=_f32) + bq_ref[...]
    kv_ref[...] = jnp.dot(x, wkv_ref[...], preferred_element_type=_f32) + bkv_ref[...]


def _proj(x, wq, bq, wkv, bkv):
    grid = (N_P // _BN,)
    return pl.pallas_call(
        _proj_body,
        grid=grid,
        in_specs=[
            pl.BlockSpec((_BN, C), lambda i: (i, 0)),
            pl.BlockSpec((C, C), lambda i: (0, 0)),
            pl.BlockSpec((1, C), lambda i: (0, 0)),
            pl.BlockSpec((C, 2 * C), lambda i: (0, 0)),
            pl.BlockSpec((1, 2 * C), lambda i: (0, 0)),
        ],
        out_specs=[
            pl.BlockSpec((_BN, C), lambda i: (i, 0)),
            pl.BlockSpec((_BN, 2 * C), lambda i: (i, 0)),
        ],
        out_shape=[
            jax.ShapeDtypeStruct((N_P, C), _f32),
            jax.ShapeDtypeStruct((N_P, 2 * C), _f32),
        ],
    )(x, wq, bq, wkv, bkv)


def _final_body(num_ref, den_ref, x_ref, wa_ref, ba_ref, beta_ref, o_ref):
    x = x_ref[...]
    tot = jnp.zeros((x.shape[0], C), _f32)
    for h in range(H):
        core, hp = h // 4, h % 4
        num = num_ref[core * 2 + hp // 2, :,
                      pl.ds((hp % 2) * 16, 16)]
        den = den_ref[core, :, pl.ds(hp, 1)]
        g = jax.nn.gelu(num / (den + 1e-30))
        tot = tot + jnp.dot(g, wa_ref[pl.ds(h * 16, 16), :],
                            preferred_element_type=_f32)
    beta = beta_ref[0, 0]
    y = beta * (tot + ba_ref[...]) + (1.0 - beta) * x
    o_ref[...] = jnp.where(y >= 0, y, 0.01 * y)


def _finalize(num, den, x_prev, wa, ba, beta):
    grid = (N_P // _BN,)
    return pl.pallas_call(
        _final_body,
        grid=grid,
        in_specs=[
            pl.BlockSpec((4, _BN, 32), lambda i: (0, i, 0)),
            pl.BlockSpec((2, _BN, 16), lambda i: (0, i, 0)),
            pl.BlockSpec((_BN, C), lambda i: (i, 0)),
            pl.BlockSpec((C, C), lambda i: (0, 0)),
            pl.BlockSpec((1, C), lambda i: (0, 0)),
            pl.BlockSpec((1, 1), lambda i: (0, 0), memory_space=pltpu.SMEM),
        ],
        out_specs=pl.BlockSpec((_BN, C), lambda i: (i, 0)),
        out_shape=jax.ShapeDtypeStruct((N_P, C), _f32),
    )(num, den, x_prev, wa, ba, beta)


def _fit_body(er_ref, el_ref, w2_ref, b2_ref, emb_ref, fit_ref):
    emb = er_ref[...] + el_ref[...]
    emb_ref[...] = emb
    fit_ref[...] = (jnp.sum(emb * w2_ref[...], axis=1, keepdims=True)
                    + b2_ref[0, 0])


def _fit(emb_res, emb_link, w2row, b2):
    grid = (N_P // _BN,)
    return pl.pallas_call(
        _fit_body,
        grid=grid,
        in_specs=[
            pl.BlockSpec((_BN, C), lambda i: (i, 0)),
            pl.BlockSpec((_BN, C), lambda i: (i, 0)),
            pl.BlockSpec((1, C), lambda i: (0, 0)),
            pl.BlockSpec((1, 1), lambda i: (0, 0), memory_space=pltpu.SMEM),
        ],
        out_specs=[
            pl.BlockSpec((_BN, C), lambda i: (i, 0)),
            pl.BlockSpec((_BN, 1), lambda i: (i, 0)),
        ],
        out_shape=[
            jax.ShapeDtypeStruct((N_P, C), _f32),
            jax.ShapeDtypeStruct((N_P, 1), _f32),
        ],
    )(emb_res, emb_link, w2row, b2)


def _score_body(zs_ref, zd_ref, ws_ref, bs_ref, wd_ref, bd_ref, bf_ref, o_ref):
    hs = jnp.dot(zs_ref[...], ws_ref[...], preferred_element_type=_f32) + bs_ref[...]
    hd = jnp.dot(zd_ref[...], wd_ref[...], preferred_element_type=_f32) + bd_ref[...]
    pre = jnp.sum(hs * hd, axis=1, keepdims=True) + bf_ref[0, 0]
    o_ref[...] = jax.nn.sigmoid(pre)


def _score(zs, zd, ws, bs, wd, bd, bf):
    m = 2 * PE
    bm = 2048
    grid = (m // bm,)
    return pl.pallas_call(
        _score_body,
        grid=grid,
        in_specs=[
            pl.BlockSpec((bm, C), lambda i: (i, 0)),
            pl.BlockSpec((bm, C), lambda i: (i, 0)),
            pl.BlockSpec((C, 2 * C), lambda i: (0, 0)),
            pl.BlockSpec((1, 2 * C), lambda i: (0, 0)),
            pl.BlockSpec((C, 2 * C), lambda i: (0, 0)),
            pl.BlockSpec((1, 2 * C), lambda i: (0, 0)),
            pl.BlockSpec((1, 1), lambda i: (0, 0), memory_space=pltpu.SMEM),
        ],
        out_specs=pl.BlockSpec((bm, 1), lambda i: (i, 0)),
        out_shape=jax.ShapeDtypeStruct((m, 1), _f32),
    )(zs, zd, ws, bs, wd, bd, bf)


def _fold_layer(p):
    out = {}
    for t, src_e, dst_e in (('dev', 'dev__repo', 'repo__dev'),
                            ('repo', 'repo__dev', 'dev__repo')):
        npar = p[t]
        scale = p[dst_e]['p'] / 4.0
        wq = (npar['Wq'].reshape(C, H, Dh) * scale[None, :, None]).reshape(C, C)
        bq = (npar['bq'].reshape(H, Dh) * scale[:, None]).reshape(1, C)
        att, msg = p[src_e]['att'], p[src_e]['msg']
        wk = jnp.einsum('chd,hde->che', npar['Wk'].reshape(C, H, Dh), att)
        bk = jnp.einsum('hd,hde->he', npar['bk'].reshape(H, Dh), att)
        wv = jnp.einsum('chd,hde->che', npar['Wv'].reshape(C, H, Dh), msg)
        bv = jnp.einsum('hd,hde->he', npar['bv'].reshape(H, Dh), msg)
        wkv = jnp.concatenate([wk, wv], axis=-1).reshape(C, 2 * C)
        bkv = jnp.concatenate([bk, bv], axis=-1).reshape(1, 2 * C)
        out[t] = dict(Wq=wq, bq=bq, Wkv=wkv, bkv=bkv, Wa=npar['Wa'],
                      ba=npar['ba'].reshape(1, C),
                      beta=jax.nn.sigmoid(npar['skip']).reshape(1, 1))
    return out


def kernel(x_dev, x_repo, edge_index_dev_repo, edge_index_repo_dev,
           pos_edge, neg_edge, params):
    pad = _EPAD - E
    s_dr = jnp.pad(edge_index_dev_repo[0], (0, pad)).reshape(-1, 128)
    d_dr = jnp.pad(edge_index_dev_repo[1], (0, pad)).reshape(-1, 128)
    s_rd = jnp.pad(edge_index_repo_dev[0], (0, pad)).reshape(-1, 128)
    d_rd = jnp.pad(edge_index_repo_dev[1], (0, pad)).reshape(-1, 128)

    xp_dev = jnp.pad(x_dev, ((0, N_P - N), (0, 0)))
    xp_repo = jnp.pad(x_repo, ((0, N_P - N), (0, 0)))

    def net(layers):
        xd = {'dev': xp_dev, 'repo': xp_repo}
        for p in layers:
            f = _fold_layer(p)
            q, kv = {}, {}
            for t in ('dev', 'repo'):
                qt, kvt = _proj(xd[t], f[t]['Wq'], f[t]['bq'],
                                f[t]['Wkv'], f[t]['bkv'])
                q[t] = qt.reshape(N_P * 2, 4 * Dh)
                kv[t] = kvt.reshape(N_P * 2, 8 * Dh)
            msg_r, den_repo = _edgeA(q['repo'], kv['dev'], s_dr, d_dr)
            num_repo = _edgeB(msg_r, d_dr)
            msg_d, den_dev = _edgeA(q['dev'], kv['repo'], s_rd, d_rd)
            num_dev = _edgeB(msg_d, d_rd)
            xd = {'dev': _finalize(num_dev, den_dev, xd['dev'],
                                   f['dev']['Wa'], f['dev']['ba'],
                                   f['dev']['beta']),
                  'repo': _finalize(num_repo, den_repo, xd['repo'],
                                    f['repo']['Wa'], f['repo']['ba'],
                                    f['repo']['beta'])}
        return xd

    er = net(params['res'])
    el = net(params['link'])

    reg = params['reg']
    w2 = (reg['lin']['W'] @ reg['regress']['W']).reshape(1, C)
    b2 = (reg['lin']['b'] @ reg['regress']['W']
          + reg['regress']['b']).reshape(1, 1)
    emb_dev, fit_dev = _fit(er['dev'], el['dev'], w2, b2)
    emb_repo, fit_repo = _fit(er['repo'], el['repo'], w2, b2)

    idx_dev = jnp.concatenate([pos_edge[0], neg_edge[0]]).reshape(128, 128)
    idx_repo = jnp.concatenate([pos_edge[1], neg_edge[1]]).reshape(128, 128)
    zs, zd = _lp_gather(emb_dev, emb_repo, idx_dev, idx_repo)

    lp = params['lp']
    wf = lp['final']['W'][:, 0]
    ws = lp['src']['W'] * wf[None, :]
    bs = (lp['src']['b'] * wf).reshape(1, 2 * C)
    wd = lp['dst']['W']
    bd = lp['dst']['b'].reshape(1, 2 * C)
    bf = lp['final']['b'].reshape(1, 1)
    score = _score(zs, zd, ws, bs, wd, bd, bf)

    return fit_dev[:N, 0], fit_repo[:N, 0], score[:, 0]
